# baseline (speedup 1.0000x reference)
import jax
import jax.numpy as jnp
from jax.experimental import pallas as pl

_BANK = 16
_BATCH = 32
_ROWS = 8


def _combine_body(idx_ref, prob_ref, p_ref, o_ref):
    idx = idx_ref[...]
    prob = prob_ref[...]
    e = jax.lax.broadcasted_iota(jnp.int32, (1, 1, _BANK), 2)
    onehot = (idx[:, :, None] == e).astype(jnp.float32)
    w = jnp.sum(prob[:, :, None] * onehot, axis=1)
    for r in range(_ROWS):
        o_ref[:, r, :] = jax.lax.dot_general(
            w, p_ref[r], (((1,), (1,)), ((), ())),
            preferred_element_type=jnp.float32)


def kernel(parameter, selection_index, selection_probabilities):
    h, w_dim, bank = parameter.shape
    out = pl.pallas_call(
        _combine_body,
        grid=(h // _ROWS,),
        in_specs=[
            pl.BlockSpec((_BATCH, 2), lambda i: (0, 0)),
            pl.BlockSpec((_BATCH, 2), lambda i: (0, 0)),
            pl.BlockSpec((_ROWS, w_dim, bank), lambda i: (i, 0, 0)),
        ],
        out_specs=pl.BlockSpec((_BATCH, _ROWS, w_dim), lambda i: (0, i, 0)),
        out_shape=jax.ShapeDtypeStruct((_BATCH, h, w_dim), jnp.float32),
    )(selection_index, selection_probabilities, parameter)
    return out


# speedup vs baseline: 3.6114x; 3.6114x over previous
import jax
import jax.numpy as jnp
from jax.experimental import pallas as pl

_BANK = 16
_BATCH = 32
_ROWS = 8


def _combine_body(idx_ref, prob_ref, p_ref, o_ref):
    idx = idx_ref[...]
    prob = prob_ref[...]
    e = jax.lax.broadcasted_iota(jnp.int32, (1, 1, _BANK), 2)
    onehot = (idx[:, :, None] == e).astype(jnp.float32)
    w = jnp.sum(prob[:, :, None] * onehot, axis=1)
    for r in range(_ROWS):
        seg = p_ref[pl.ds(r * 1024, 1024), :]
        o_ref[:, r, :] = jax.lax.dot_general(
            w, seg, (((1,), (1,)), ((), ())),
            preferred_element_type=jnp.float32)


def kernel(parameter, selection_index, selection_probabilities):
    h, w_dim, bank = parameter.shape
    m = h * w_dim
    pf = parameter.reshape(m, bank)
    out = pl.pallas_call(
        _combine_body,
        grid=(h // _ROWS,),
        in_specs=[
            pl.BlockSpec((_BATCH, 2), lambda i: (0, 0)),
            pl.BlockSpec((_BATCH, 2), lambda i: (0, 0)),
            pl.BlockSpec((_ROWS * w_dim, bank), lambda i: (i, 0)),
        ],
        out_specs=pl.BlockSpec((_BATCH, _ROWS, w_dim), lambda i: (0, i, 0)),
        out_shape=jax.ShapeDtypeStruct((_BATCH, h, w_dim), jnp.float32),
    )(selection_index, selection_probabilities, pf)
    return out


# speedup vs baseline: 6.1155x; 1.6934x over previous
import jax
import jax.numpy as jnp
from jax.experimental import pallas as pl

_BANK = 16
_BATCH = 32
_ROWS = 8


def _combine_body(idx_ref, prob_ref, p_ref, o_ref):
    idx = idx_ref[...]
    prob = prob_ref[...]
    e = jax.lax.broadcasted_iota(jnp.int32, (1, 1, _BANK), 2)
    onehot = (idx[:, :, None] == e).astype(jnp.float32)
    w = jnp.sum(prob[:, :, None] * onehot, axis=1)
    for r in range(_ROWS):
        o_ref[:, r, :] = jax.lax.dot_general(
            w, p_ref[r], (((1,), (0,)), ((), ())),
            preferred_element_type=jnp.float32)


def kernel(parameter, selection_index, selection_probabilities):
    h, w_dim, bank = parameter.shape
    p_t = jnp.transpose(parameter, (0, 2, 1))
    out = pl.pallas_call(
        _combine_body,
        grid=(h // _ROWS,),
        in_specs=[
            pl.BlockSpec((_BATCH, 2), lambda i: (0, 0)),
            pl.BlockSpec((_BATCH, 2), lambda i: (0, 0)),
            pl.BlockSpec((_ROWS, bank, w_dim), lambda i: (i, 0, 0)),
        ],
        out_specs=pl.BlockSpec((_BATCH, _ROWS, w_dim), lambda i: (0, i, 0)),
        out_shape=jax.ShapeDtypeStruct((_BATCH, h, w_dim), jnp.float32),
    )(selection_index, selection_probabilities, p_t)
    return out
